# Initial kernel scaffold; baseline (speedup 1.0000x reference)
import jax
import jax.numpy as jnp
from jax.experimental import pallas as pl


def kernel(trigger):
    raise NotImplementedError("write your pallas kernel here")


# speedup vs baseline: 23.5958x; 23.5958x over previous
import jax
import jax.numpy as jnp
from jax import lax
from jax.experimental import pallas as pl
from jax.experimental.pallas import tpu as pltpu

_ROWS, _COLS = 4096, 1024
_N = _ROWS * _COLS
_KS = (int(0.05 * _N), int(0.1 * _N), int(0.2 * _N))
_ONE_BITS = 0x3F800000
_VAL_ITERS = 30
_IDX_ITERS = 23


def _threshold_kernel(x_ref, thr_ref, cut_ref):
    u = lax.bitcast_convert_type(x_ref[...], jnp.int32)

    def count_ge(t):
        return jnp.sum((u >= t).astype(jnp.int32))

    def val_step(_, c):
        out = []
        for s in range(3):
            lo, hi, ghi = c[3 * s], c[3 * s + 1], c[3 * s + 2]
            mid = (lo + hi) >> 1
            cnt = count_ge(mid)
            ok = cnt >= _KS[s]
            out += [jnp.where(ok, mid, lo),
                    jnp.where(ok, hi, mid),
                    jnp.where(ok, ghi, cnt)]
        return tuple(out)

    z = jnp.int32(0)
    top = jnp.int32(_ONE_BITS)
    c = lax.fori_loop(0, _VAL_ITERS, val_step, (z, top, z) * 3)

    fidx = (lax.broadcasted_iota(jnp.int32, (_ROWS, _COLS), 0) * _COLS
            + lax.broadcasted_iota(jnp.int32, (_ROWS, _COLS), 1))

    for s in range(3):
        t_s, cnt_gt = c[3 * s], c[3 * s + 2]
        need = jnp.int32(_KS[s]) - cnt_gt
        eq = u == t_s

        def idx_step(_, ci, eq=eq, need=need):
            lo, hi = ci
            mid = (lo + hi) >> 1
            cnt = jnp.sum((eq & (fidx <= mid)).astype(jnp.int32))
            ok = cnt >= need
            return (jnp.where(ok, lo, mid), jnp.where(ok, mid, hi))

        _, cut = lax.fori_loop(0, _IDX_ITERS, idx_step,
                               (jnp.int32(-1), jnp.int32(_N - 1)))
        thr_ref[s] = t_s
        cut_ref[s] = cut


_MBLK = 256


def _mask_kernel(thr_ref, cut_ref, x_ref, o_ref):
    i = pl.program_id(0)
    v = x_ref[...]
    u = lax.bitcast_convert_type(v, jnp.int32)
    fidx = ((i * _MBLK + lax.broadcasted_iota(jnp.int32, (_MBLK, _COLS), 0))
            * _COLS
            + lax.broadcasted_iota(jnp.int32, (_MBLK, _COLS), 1))
    for s in range(3):
        drop = (u > thr_ref[s]) | ((u == thr_ref[s]) & (fidx <= cut_ref[s]))
        o_ref[s] = jnp.where(drop, 0.0, v)


def kernel(trigger):
    thr, cut = pl.pallas_call(
        _threshold_kernel,
        out_shape=(jax.ShapeDtypeStruct((3,), jnp.int32),
                   jax.ShapeDtypeStruct((3,), jnp.int32)),
        in_specs=[pl.BlockSpec(memory_space=pltpu.VMEM)],
        out_specs=(pl.BlockSpec(memory_space=pltpu.SMEM),
                   pl.BlockSpec(memory_space=pltpu.SMEM)),
    )(trigger)

    out = pl.pallas_call(
        _mask_kernel,
        grid=(_ROWS // _MBLK,),
        out_shape=jax.ShapeDtypeStruct((3, _ROWS, _COLS), jnp.float32),
        in_specs=[
            pl.BlockSpec(memory_space=pltpu.SMEM),
            pl.BlockSpec(memory_space=pltpu.SMEM),
            pl.BlockSpec((_MBLK, _COLS), lambda i: (i, 0)),
        ],
        out_specs=pl.BlockSpec((3, _MBLK, _COLS), lambda i: (0, i, 0)),
    )(thr, cut, trigger)
    return out
